# Initial kernel scaffold; baseline (speedup 1.0000x reference)
import functools

import jax
import jax.numpy as jnp
from jax.experimental import pallas as pl
from jax.experimental.pallas import tpu as pltpu


def _gating_body(u_ref, c_ref, eb_ref, g_ref):
    scores = jax.lax.dot_general(
        u_ref[...], c_ref[...], (((1,), (1,)), ((), ())),
        preferred_element_type=jnp.float32)
    scores = scores + eb_ref[...]
    m = jnp.max(scores, axis=-1, keepdims=True)
    ex = jnp.exp(scores - m)
    s = ex / jnp.sum(ex, axis=-1, keepdims=True)
    nr = s.shape[-1]
    iota = jax.lax.broadcasted_iota(jnp.int32, s.shape, 1)
    e0 = jnp.argmax(s, axis=-1)
    oh0 = iota == e0[:, None]
    s_m = jnp.where(oh0, -jnp.inf, s)
    e1 = jnp.argmax(s_m, axis=-1)
    oh1 = iota == e1[:, None]
    gtop = jnp.where(oh0 | oh1, s, 0.0)
    mt = gtop.shape[0]
    pad = g_ref.shape[-1] - nr
    shared_w = jnp.full((mt, 2), 0.5, jnp.float32)
    zeros = jnp.zeros((mt, pad - 2), jnp.float32)
    g_ref[...] = jnp.concatenate([gtop, shared_w, zeros], axis=-1)


def _moe_body(u_ref, w_ref, b_ref, g_ref, o_ref, acc_ref, *, num_e):
    e = pl.program_id(1)

    @pl.when(e == 0)
    def _init():
        acc_ref[...] = u_ref[...]

    y = jax.lax.dot_general(
        u_ref[...], w_ref[0], (((1,), (1,)), ((), ())),
        preferred_element_type=jnp.float32)
    mask = jax.lax.broadcasted_iota(jnp.int32, g_ref.shape, 1) == e
    gcol = jnp.sum(jnp.where(mask, g_ref[...], 0.0), axis=1, keepdims=True)
    acc_ref[...] += gcol * (y + b_ref[...])

    @pl.when(e == num_e - 1)
    def _fin():
        o_ref[...] = acc_ref[...]


def kernel(u, centroids, expert_biases, W_r, b_r, W_s, b_s):
    B, S, D = u.shape
    NR = W_r.shape[0]
    NS = W_s.shape[0]
    T = B * S
    NE = NR + NS

    u2 = u.reshape(T, D)
    Wc = jnp.concatenate([W_r, W_s], axis=0)
    bc = jnp.concatenate([b_r, b_s], axis=0)

    MT_G = min(1024, T)
    g = pl.pallas_call(
        _gating_body,
        grid=(T // MT_G,),
        in_specs=[
            pl.BlockSpec((MT_G, D), lambda i: (i, 0)),
            pl.BlockSpec((NR, D), lambda i: (0, 0)),
            pl.BlockSpec((1, NR), lambda i: (0, 0)),
        ],
        out_specs=pl.BlockSpec((MT_G, 16), lambda i: (i, 0)),
        out_shape=jax.ShapeDtypeStruct((T, 16), jnp.float32),
    )(u2, centroids, expert_biases.reshape(1, NR))

    MT = min(512, T)
    h = pl.pallas_call(
        functools.partial(_moe_body, num_e=NE),
        grid=(T // MT, NE),
        in_specs=[
            pl.BlockSpec((MT, D), lambda i, e: (i, 0)),
            pl.BlockSpec((1, D, D), lambda i, e: (e, 0, 0)),
            pl.BlockSpec((1, D), lambda i, e: (e, 0)),
            pl.BlockSpec((MT, 16), lambda i, e: (i, 0)),
        ],
        out_specs=pl.BlockSpec((MT, D), lambda i, e: (i, 0)),
        out_shape=jax.ShapeDtypeStruct((T, D), jnp.float32),
        scratch_shapes=[pltpu.VMEM((MT, D), jnp.float32)],
    )(u2, Wc, bc, g)

    return h.reshape(B, S, D)


# speedup vs baseline: 1.3895x; 1.3895x over previous
import functools

import jax
import jax.numpy as jnp
from jax.experimental import pallas as pl
from jax.experimental.pallas import tpu as pltpu


def _gating_body(u_ref, c_ref, eb_ref, g_ref):
    scores = jax.lax.dot_general(
        u_ref[...], c_ref[...], (((1,), (1,)), ((), ())),
        preferred_element_type=jnp.float32)
    scores = scores + eb_ref[...]
    m = jnp.max(scores, axis=-1, keepdims=True)
    ex = jnp.exp(scores - m)
    s = ex / jnp.sum(ex, axis=-1, keepdims=True)
    nr = s.shape[-1]
    iota = jax.lax.broadcasted_iota(jnp.int32, s.shape, 1)
    e0 = jnp.argmax(s, axis=-1)
    oh0 = iota == e0[:, None]
    s_m = jnp.where(oh0, -jnp.inf, s)
    e1 = jnp.argmax(s_m, axis=-1)
    oh1 = iota == e1[:, None]
    gtop = jnp.where(oh0 | oh1, s, 0.0)
    mt = gtop.shape[0]
    pad = g_ref.shape[-1] - nr
    shared_w = jnp.full((mt, 2), 0.5, jnp.float32)
    zeros = jnp.zeros((mt, pad - 2), jnp.float32)
    g_ref[...] = jnp.concatenate([gtop, shared_w, zeros], axis=-1)


def _moe_body(u_ref, w_ref, b_ref, g_ref, o_ref, acc_ref, *, num_e):
    e = pl.program_id(1)

    @pl.when(e == 0)
    def _init():
        acc_ref[...] = u_ref[...]

    y = jax.lax.dot_general(
        u_ref[...], w_ref[0], (((1,), (1,)), ((), ())),
        preferred_element_type=jnp.float32)
    mask = jax.lax.broadcasted_iota(jnp.int32, g_ref.shape, 1) == e
    gcol = jnp.sum(jnp.where(mask, g_ref[...], 0.0), axis=1, keepdims=True)
    acc_ref[...] += gcol * (y + b_ref[0])

    @pl.when(e == num_e - 1)
    def _fin():
        o_ref[...] = acc_ref[...]


def kernel(u, centroids, expert_biases, W_r, b_r, W_s, b_s):
    B, S, D = u.shape
    NR = W_r.shape[0]
    NS = W_s.shape[0]
    T = B * S
    NE = NR + NS

    u2 = u.reshape(T, D)
    Wc = jnp.concatenate([W_r, W_s], axis=0)
    bc = jnp.concatenate([b_r, b_s], axis=0).reshape(NE, 1, D)

    MT_G = min(1024, T)
    g = pl.pallas_call(
        _gating_body,
        grid=(T // MT_G,),
        in_specs=[
            pl.BlockSpec((MT_G, D), lambda i: (i, 0)),
            pl.BlockSpec((NR, D), lambda i: (0, 0)),
            pl.BlockSpec((1, NR), lambda i: (0, 0)),
        ],
        out_specs=pl.BlockSpec((MT_G, 16), lambda i: (i, 0)),
        out_shape=jax.ShapeDtypeStruct((T, 16), jnp.float32),
    )(u2, centroids, expert_biases.reshape(1, NR))

    MT = min(512, T)
    h = pl.pallas_call(
        functools.partial(_moe_body, num_e=NE),
        grid=(T // MT, NE),
        in_specs=[
            pl.BlockSpec((MT, D), lambda i, e: (i, 0)),
            pl.BlockSpec((1, D, D), lambda i, e: (e, 0, 0)),
            pl.BlockSpec((1, 1, D), lambda i, e: (e, 0, 0)),
            pl.BlockSpec((MT, 16), lambda i, e: (i, 0)),
        ],
        out_specs=pl.BlockSpec((MT, D), lambda i, e: (i, 0)),
        out_shape=jax.ShapeDtypeStruct((T, D), jnp.float32),
        scratch_shapes=[pltpu.VMEM((MT, D), jnp.float32)],
    )(u2, Wc, bc, g)

    return h.reshape(B, S, D)


# speedup vs baseline: 1.4486x; 1.0425x over previous
import functools

import jax
import jax.numpy as jnp
from jax import lax
from jax.experimental import pallas as pl
from jax.experimental.pallas import tpu as pltpu
from jax.experimental.pallas import tpu_sc as plsc

HI = jax.lax.Precision.HIGHEST


def _gating_body(u_ref, c_ref, eb_ref, us_ref, oh_ref, wh_ref):
    sl = pl.program_id(1)
    scores = jax.lax.dot_general(
        u_ref[...], c_ref[...], (((1,), (1,)), ((), ())),
        preferred_element_type=jnp.float32)
    scores = scores + eb_ref[...]
    m = jnp.max(scores, axis=-1, keepdims=True)
    ex = jnp.exp(scores - m)
    s = ex / jnp.sum(ex, axis=-1, keepdims=True)
    iota = jax.lax.broadcasted_iota(jnp.int32, s.shape, 1)
    e0 = jnp.argmax(s, axis=-1)
    oh0 = iota == e0[:, None]
    e1 = jnp.argmax(jnp.where(oh0, -jnp.inf, s), axis=-1)
    oh1 = iota == e1[:, None]
    ohf = jnp.where(sl == 0, oh0.astype(jnp.float32), oh1.astype(jnp.float32))
    wh = s * ohf
    w = jnp.sum(wh, axis=-1, keepdims=True)
    us_ref[...] = w * u_ref[...]
    oh_ref[...] = ohf
    wh_ref[...] = wh


def _routing_body(oh_ref, dest_ref, bexp_ref, *, blk_m, nblocks):
    P, NR = oh_ref.shape
    CH = 512
    nch = P // CH
    r_i = jax.lax.broadcasted_iota(jnp.int32, (CH, CH), 0)
    c_i = jax.lax.broadcasted_iota(jnp.int32, (CH, CH), 1)
    tri = (r_i >= c_i).astype(jnp.float32)
    carry = jnp.zeros((1, NR), jnp.float32)
    cums = []
    for k in range(nch):
        blk = oh_ref[pl.ds(k * CH, CH), :]
        cb = jax.lax.dot_general(
            tri, blk, (((1,), (0,)), ((), ())),
            precision=HI, preferred_element_type=jnp.float32) + carry
        cums.append(cb)
        carry = cb[CH - 1:CH, :]
    counts = carry
    pc = jnp.ceil(counts * (1.0 / blk_m)) * blk_m
    a_i = jax.lax.broadcasted_iota(jnp.int32, (NR, NR), 0)
    b_i = jax.lax.broadcasted_iota(jnp.int32, (NR, NR), 1)
    lt = (a_i < b_i).astype(jnp.float32)
    off = jax.lax.dot_general(
        pc, lt, (((1,), (0,)), ((), ())),
        precision=HI, preferred_element_type=jnp.float32)
    for k in range(nch):
        ohk = oh_ref[pl.ds(k * CH, CH), :]
        destf = jnp.sum(ohk * (off + cums[k] - 1.0), axis=-1, keepdims=True)
        dest_ref[pl.ds(k * CH, CH), :] = destf.astype(jnp.int32)
    offi = off + pc
    lane = jax.lax.broadcasted_iota(
        jnp.int32, bexp_ref.shape, 1).astype(jnp.float32) * blk_m
    acc = jnp.zeros(bexp_ref.shape, jnp.float32)
    for e in range(NR):
        acc += jnp.where(lane >= offi[0, e], 1.0, 0.0)
    bexp_ref[...] = jnp.minimum(acc, NR - 1).astype(jnp.int32)


def _gmm_body(bexp_sref, x_ref, w_ref, o_ref):
    del bexp_sref
    o_ref[...] = jax.lax.dot_general(
        x_ref[...], w_ref[0], (((1,), (1,)), ((), ())),
        preferred_element_type=jnp.float32)


def _base_body(u_ref, wsm_ref, bsm_ref, wh0_ref, wh1_ref, br_ref, o_ref):
    ub = u_ref[...]
    y = jax.lax.dot_general(
        ub, wsm_ref[...], (((1,), (1,)), ((), ())),
        preferred_element_type=jnp.float32)
    whs = wh0_ref[...] + wh1_ref[...]
    bias = jax.lax.dot_general(
        whs, br_ref[...], (((1,), (0,)), ((), ())),
        precision=HI, preferred_element_type=jnp.float32)
    o_ref[...] = ub + y + bsm_ref[...] + bias


def _sc_dispatch(us, dest3d, R):
    P, D = us.shape
    NW, C, G = dest3d.shape
    ppw = P // NW
    mesh = plsc.VectorSubcoreMesh(core_axis_name="c", subcore_axis_name="s")

    @functools.partial(
        pl.kernel, mesh=mesh,
        out_type=jax.ShapeDtypeStruct((R, D), jnp.float32),
        scratch_types=[
            pltpu.VMEM((C, G), jnp.int32),
            pltpu.VMEM((G, D), jnp.float32),
            pltpu.SemaphoreType.DMA,
        ])
    def k(us_hbm, dest_hbm, out_hbm, idx_v, rows_v, sem):
        wid = lax.axis_index("s") * 2 + lax.axis_index("c")
        pltpu.sync_copy(dest_hbm.at[wid], idx_v)
        base = wid * ppw
        for j in range(C):
            pltpu.sync_copy(us_hbm.at[pl.ds(base + j * G, G)], rows_v)
            pltpu.async_copy(rows_v, out_hbm.at[idx_v.at[j]], sem).wait()

    return k(us, dest3d)


def _sc_combine(base, rows, pos0_3d, pos1_3d):
    T, D = base.shape
    NW, C2, G2 = pos0_3d.shape
    tpw = T // NW
    mesh = plsc.VectorSubcoreMesh(core_axis_name="c", subcore_axis_name="s")

    @functools.partial(
        pl.kernel, mesh=mesh,
        out_type=jax.ShapeDtypeStruct((T, D), jnp.float32),
        scratch_types=[
            pltpu.VMEM((C2, G2), jnp.int32),
            pltpu.VMEM((C2, G2), jnp.int32),
            pltpu.VMEM((G2, D), jnp.float32),
            pltpu.VMEM((G2, D), jnp.float32),
            pltpu.VMEM((G2, D), jnp.float32),
            pltpu.SemaphoreType.DMA,
        ])
    def k(base_hbm, rows_hbm, p0_hbm, p1_hbm, out_hbm,
          i0_v, i1_v, b_v, r0_v, r1_v, sem):
        wid = lax.axis_index("s") * 2 + lax.axis_index("c")
        pltpu.sync_copy(p0_hbm.at[wid], i0_v)
        pltpu.sync_copy(p1_hbm.at[wid], i1_v)
        tb = wid * tpw
        for j in range(C2):
            pltpu.sync_copy(base_hbm.at[pl.ds(tb + j * G2, G2)], b_v)
            pltpu.async_copy(rows_hbm.at[i0_v.at[j]], r0_v, sem).wait()
            pltpu.async_copy(rows_hbm.at[i1_v.at[j]], r1_v, sem).wait()

            def row(i, car):
                def col(kk, car2):
                    sl = pl.ds(kk * 16, 16)
                    b_v[i, sl] = b_v[i, sl] + r0_v[i, sl] + r1_v[i, sl]
                    return car2
                return lax.fori_loop(0, D // 16, col, car, unroll=8)

            lax.fori_loop(0, G2, row, 0)
            pltpu.sync_copy(b_v, out_hbm.at[pl.ds(tb + j * G2, G2)])

    return k(base, rows, pos0_3d, pos1_3d)


def kernel(u, centroids, expert_biases, W_r, b_r, W_s, b_s):
    B, S, D = u.shape
    NR = W_r.shape[0]
    T = B * S
    P = 2 * T
    M = 256
    NB = P // M + NR
    R = NB * M
    NW = 32
    G, G2 = 32, 16
    C = P // NW // G
    C2 = T // NW // G2

    u2 = u.reshape(T, D)
    Wsm = jnp.mean(W_s, axis=0)
    bsm = jnp.mean(b_s, axis=0).reshape(1, D)

    MTg = 512
    ngt = T // MTg
    us, ohp, whp = pl.pallas_call(
        _gating_body,
        grid=(ngt, 2),
        in_specs=[
            pl.BlockSpec((MTg, D), lambda i, sl: (i, 0)),
            pl.BlockSpec((NR, D), lambda i, sl: (0, 0)),
            pl.BlockSpec((1, NR), lambda i, sl: (0, 0)),
        ],
        out_specs=[
            pl.BlockSpec((MTg, D), lambda i, sl: (sl * ngt + i, 0)),
            pl.BlockSpec((MTg, NR), lambda i, sl: (sl * ngt + i, 0)),
            pl.BlockSpec((MTg, NR), lambda i, sl: (sl * ngt + i, 0)),
        ],
        out_shape=[
            jax.ShapeDtypeStruct((P, D), jnp.float32),
            jax.ShapeDtypeStruct((P, NR), jnp.float32),
            jax.ShapeDtypeStruct((P, NR), jnp.float32),
        ],
    )(u2, centroids, expert_biases.reshape(1, NR))

    dest, bexp = pl.pallas_call(
        functools.partial(_routing_body, blk_m=M, nblocks=NB),
        grid=(1,),
        in_specs=[pl.BlockSpec((P, NR), lambda i: (0, 0))],
        out_specs=[
            pl.BlockSpec((P, 1), lambda i: (0, 0)),
            pl.BlockSpec((1, 128), lambda i: (0, 0)),
        ],
        out_shape=[
            jax.ShapeDtypeStruct((P, 1), jnp.int32),
            jax.ShapeDtypeStruct((1, 128), jnp.int32),
        ],
    )(ohp)

    dest1 = dest.reshape(P)
    dest3d = dest1.reshape(NW, C, G)
    pos0_3d = dest1[:T].reshape(NW, C2, G2)
    pos1_3d = dest1[T:].reshape(NW, C2, G2)
    bexp_flat = bexp.reshape(128)[:NB]

    xd = _sc_dispatch(us, dest3d, R)

    MTb = 512
    base = pl.pallas_call(
        _base_body,
        grid=(T // MTb,),
        in_specs=[
            pl.BlockSpec((MTb, D), lambda i: (i, 0)),
            pl.BlockSpec((D, D), lambda i: (0, 0)),
            pl.BlockSpec((1, D), lambda i: (0, 0)),
            pl.BlockSpec((MTb, NR), lambda i: (i, 0)),
            pl.BlockSpec((MTb, NR), lambda i: (i + T // MTb, 0)),
            pl.BlockSpec((NR, D), lambda i: (0, 0)),
        ],
        out_specs=pl.BlockSpec((MTb, D), lambda i: (i, 0)),
        out_shape=jax.ShapeDtypeStruct((T, D), jnp.float32),
    )(u2, Wsm, bsm, whp, whp, b_r)

    outs = pl.pallas_call(
        _gmm_body,
        grid_spec=pltpu.PrefetchScalarGridSpec(
            num_scalar_prefetch=1,
            grid=(NB,),
            in_specs=[
                pl.BlockSpec((M, D), lambda i, b: (i, 0)),
                pl.BlockSpec((1, D, D), lambda i, b: (b[i], 0, 0)),
            ],
            out_specs=pl.BlockSpec((M, D), lambda i, b: (i, 0)),
        ),
        out_shape=jax.ShapeDtypeStruct((R, D), jnp.float32),
    )(bexp_flat, xd, W_r)

    h2 = _sc_combine(base, outs, pos0_3d, pos1_3d)
    return h2.reshape(B, S, D)


# speedup vs baseline: 1.4490x; 1.0003x over previous
import functools

import jax
import jax.numpy as jnp
from jax import lax
from jax.experimental import pallas as pl
from jax.experimental.pallas import tpu as pltpu
from jax.experimental.pallas import tpu_sc as plsc

HI = jax.lax.Precision.HIGHEST


def _gating_body(u_ref, c_ref, eb_ref, us_ref, oh_ref, wh_ref):
    sl = pl.program_id(1)
    scores = jax.lax.dot_general(
        u_ref[...], c_ref[...], (((1,), (1,)), ((), ())),
        preferred_element_type=jnp.float32)
    scores = scores + eb_ref[...]
    m = jnp.max(scores, axis=-1, keepdims=True)
    ex = jnp.exp(scores - m)
    s = ex / jnp.sum(ex, axis=-1, keepdims=True)
    iota = jax.lax.broadcasted_iota(jnp.int32, s.shape, 1)
    e0 = jnp.argmax(s, axis=-1)
    oh0 = iota == e0[:, None]
    e1 = jnp.argmax(jnp.where(oh0, -jnp.inf, s), axis=-1)
    oh1 = iota == e1[:, None]
    ohf = jnp.where(sl == 0, oh0.astype(jnp.float32), oh1.astype(jnp.float32))
    wh = s * ohf
    w = jnp.sum(wh, axis=-1, keepdims=True)
    us_ref[...] = w * u_ref[...]
    oh_ref[...] = ohf
    wh_ref[...] = wh


def _routing_body(oh_ref, dest_ref, bexp_ref, *, blk_m, nblocks):
    P, NR = oh_ref.shape
    CH = 512
    nch = P // CH
    r_i = jax.lax.broadcasted_iota(jnp.int32, (CH, CH), 0)
    c_i = jax.lax.broadcasted_iota(jnp.int32, (CH, CH), 1)
    tri = (r_i >= c_i).astype(jnp.float32)
    carry = jnp.zeros((1, NR), jnp.float32)
    cums = []
    for k in range(nch):
        blk = oh_ref[pl.ds(k * CH, CH), :]
        cb = jax.lax.dot_general(
            tri, blk, (((1,), (0,)), ((), ())),
            precision=HI, preferred_element_type=jnp.float32) + carry
        cums.append(cb)
        carry = cb[CH - 1:CH, :]
    counts = carry
    pc = jnp.ceil(counts * (1.0 / blk_m)) * blk_m
    a_i = jax.lax.broadcasted_iota(jnp.int32, (NR, NR), 0)
    b_i = jax.lax.broadcasted_iota(jnp.int32, (NR, NR), 1)
    lt = (a_i < b_i).astype(jnp.float32)
    off = jax.lax.dot_general(
        pc, lt, (((1,), (0,)), ((), ())),
        precision=HI, preferred_element_type=jnp.float32)
    for k in range(nch):
        ohk = oh_ref[pl.ds(k * CH, CH), :]
        destf = jnp.sum(ohk * (off + cums[k] - 1.0), axis=-1, keepdims=True)
        dest_ref[pl.ds(k * CH, CH), :] = destf.astype(jnp.int32)
    offi = off + pc
    lane = jax.lax.broadcasted_iota(
        jnp.int32, bexp_ref.shape, 1).astype(jnp.float32) * blk_m
    acc = jnp.zeros(bexp_ref.shape, jnp.float32)
    for e in range(NR):
        acc += jnp.where(lane >= offi[0, e], 1.0, 0.0)
    bexp_ref[...] = jnp.minimum(acc, NR - 1).astype(jnp.int32)


def _gmm_body(bexp_sref, x_ref, w_ref, o_ref):
    del bexp_sref
    o_ref[...] = jax.lax.dot_general(
        x_ref[...].astype(jnp.bfloat16), w_ref[0].astype(jnp.bfloat16),
        (((1,), (1,)), ((), ())),
        preferred_element_type=jnp.float32)


def _base_body(u_ref, wsm_ref, bsm_ref, wh0_ref, wh1_ref, br_ref, o_ref):
    ub = u_ref[...]
    y = jax.lax.dot_general(
        ub.astype(jnp.bfloat16), wsm_ref[...].astype(jnp.bfloat16),
        (((1,), (1,)), ((), ())),
        preferred_element_type=jnp.float32)
    whs = wh0_ref[...] + wh1_ref[...]
    bias = jax.lax.dot_general(
        whs, br_ref[...], (((1,), (0,)), ((), ())),
        precision=HI, preferred_element_type=jnp.float32)
    o_ref[...] = ub + y + bsm_ref[...] + bias


def _sc_dispatch(us, dest3d, R):
    P, D = us.shape
    NW, C, G = dest3d.shape
    ppw = P // NW
    mesh = plsc.VectorSubcoreMesh(core_axis_name="c", subcore_axis_name="s")

    @functools.partial(
        pl.kernel, mesh=mesh,
        out_type=jax.ShapeDtypeStruct((R, D), jnp.float32),
        scratch_types=[
            pltpu.VMEM((C, G), jnp.int32),
            pltpu.VMEM((G, D), jnp.float32),
            pltpu.SemaphoreType.DMA,
        ])
    def k(us_hbm, dest_hbm, out_hbm, idx_v, rows_v, sem):
        wid = lax.axis_index("s") * 2 + lax.axis_index("c")
        pltpu.sync_copy(dest_hbm.at[wid], idx_v)
        base = wid * ppw
        for j in range(C):
            pltpu.sync_copy(us_hbm.at[pl.ds(base + j * G, G)], rows_v)
            pltpu.async_copy(rows_v, out_hbm.at[idx_v.at[j]], sem).wait()

    return k(us, dest3d)


def _sc_combine(base, rows, pos0_3d, pos1_3d):
    T, D = base.shape
    NW, C2, G2 = pos0_3d.shape
    tpw = T // NW
    mesh = plsc.VectorSubcoreMesh(core_axis_name="c", subcore_axis_name="s")

    @functools.partial(
        pl.kernel, mesh=mesh,
        out_type=jax.ShapeDtypeStruct((T, D), jnp.float32),
        scratch_types=[
            pltpu.VMEM((C2, G2), jnp.int32),
            pltpu.VMEM((C2, G2), jnp.int32),
            pltpu.VMEM((G2, D), jnp.float32),
            pltpu.VMEM((G2, D), jnp.float32),
            pltpu.VMEM((G2, D), jnp.float32),
            pltpu.SemaphoreType.DMA,
        ])
    def k(base_hbm, rows_hbm, p0_hbm, p1_hbm, out_hbm,
          i0_v, i1_v, b_v, r0_v, r1_v, sem):
        wid = lax.axis_index("s") * 2 + lax.axis_index("c")
        pltpu.sync_copy(p0_hbm.at[wid], i0_v)
        pltpu.sync_copy(p1_hbm.at[wid], i1_v)
        tb = wid * tpw
        for j in range(C2):
            pltpu.sync_copy(base_hbm.at[pl.ds(tb + j * G2, G2)], b_v)
            pltpu.async_copy(rows_hbm.at[i0_v.at[j]], r0_v, sem).wait()
            pltpu.async_copy(rows_hbm.at[i1_v.at[j]], r1_v, sem).wait()

            def row(i, car):
                def col(kk, car2):
                    sl = pl.ds(kk * 16, 16)
                    b_v[i, sl] = b_v[i, sl] + r0_v[i, sl] + r1_v[i, sl]
                    return car2
                return lax.fori_loop(0, D // 16, col, car, unroll=8)

            lax.fori_loop(0, G2, row, 0)
            pltpu.sync_copy(b_v, out_hbm.at[pl.ds(tb + j * G2, G2)])

    return k(base, rows, pos0_3d, pos1_3d)


def kernel(u, centroids, expert_biases, W_r, b_r, W_s, b_s):
    B, S, D = u.shape
    NR = W_r.shape[0]
    T = B * S
    P = 2 * T
    M = 256
    NB = P // M + NR
    R = NB * M
    NW = 32
    G, G2 = 32, 16
    C = P // NW // G
    C2 = T // NW // G2

    u2 = u.reshape(T, D)
    Wsm = jnp.mean(W_s, axis=0)
    bsm = jnp.mean(b_s, axis=0).reshape(1, D)

    MTg = 512
    ngt = T // MTg
    us, ohp, whp = pl.pallas_call(
        _gating_body,
        grid=(ngt, 2),
        in_specs=[
            pl.BlockSpec((MTg, D), lambda i, sl: (i, 0)),
            pl.BlockSpec((NR, D), lambda i, sl: (0, 0)),
            pl.BlockSpec((1, NR), lambda i, sl: (0, 0)),
        ],
        out_specs=[
            pl.BlockSpec((MTg, D), lambda i, sl: (sl * ngt + i, 0)),
            pl.BlockSpec((MTg, NR), lambda i, sl: (sl * ngt + i, 0)),
            pl.BlockSpec((MTg, NR), lambda i, sl: (sl * ngt + i, 0)),
        ],
        out_shape=[
            jax.ShapeDtypeStruct((P, D), jnp.float32),
            jax.ShapeDtypeStruct((P, NR), jnp.float32),
            jax.ShapeDtypeStruct((P, NR), jnp.float32),
        ],
    )(u2, centroids, expert_biases.reshape(1, NR))

    dest, bexp = pl.pallas_call(
        functools.partial(_routing_body, blk_m=M, nblocks=NB),
        grid=(1,),
        in_specs=[pl.BlockSpec((P, NR), lambda i: (0, 0))],
        out_specs=[
            pl.BlockSpec((P, 1), lambda i: (0, 0)),
            pl.BlockSpec((1, 128), lambda i: (0, 0)),
        ],
        out_shape=[
            jax.ShapeDtypeStruct((P, 1), jnp.int32),
            jax.ShapeDtypeStruct((1, 128), jnp.int32),
        ],
    )(ohp)

    dest1 = dest.reshape(P)
    dest3d = dest1.reshape(NW, C, G)
    pos0_3d = dest1[:T].reshape(NW, C2, G2)
    pos1_3d = dest1[T:].reshape(NW, C2, G2)
    bexp_flat = bexp.reshape(128)[:NB]

    xd = _sc_dispatch(us, dest3d, R)

    MTb = 512
    base = pl.pallas_call(
        _base_body,
        grid=(T // MTb,),
        in_specs=[
            pl.BlockSpec((MTb, D), lambda i: (i, 0)),
            pl.BlockSpec((D, D), lambda i: (0, 0)),
            pl.BlockSpec((1, D), lambda i: (0, 0)),
            pl.BlockSpec((MTb, NR), lambda i: (i, 0)),
            pl.BlockSpec((MTb, NR), lambda i: (i + T // MTb, 0)),
            pl.BlockSpec((NR, D), lambda i: (0, 0)),
        ],
        out_specs=pl.BlockSpec((MTb, D), lambda i: (i, 0)),
        out_shape=jax.ShapeDtypeStruct((T, D), jnp.float32),
    )(u2, Wsm, bsm, whp, whp, b_r)

    outs = pl.pallas_call(
        _gmm_body,
        grid_spec=pltpu.PrefetchScalarGridSpec(
            num_scalar_prefetch=1,
            grid=(NB,),
            in_specs=[
                pl.BlockSpec((M, D), lambda i, b: (i, 0)),
                pl.BlockSpec((1, D, D), lambda i, b: (b[i], 0, 0)),
            ],
            out_specs=pl.BlockSpec((M, D), lambda i, b: (i, 0)),
        ),
        out_shape=jax.ShapeDtypeStruct((R, D), jnp.float32),
    )(bexp_flat, xd, W_r)

    h2 = _sc_combine(base, outs, pos0_3d, pos1_3d)
    return h2.reshape(B, S, D)


# speedup vs baseline: 1.5737x; 1.0861x over previous
import functools

import jax
import jax.numpy as jnp
from jax import lax
from jax.experimental import pallas as pl
from jax.experimental.pallas import tpu as pltpu
from jax.experimental.pallas import tpu_sc as plsc

HI = jax.lax.Precision.HIGHEST


def _gating_body(u_ref, c_ref, eb_ref, us_ref, oh_ref, wh_ref):
    sl = pl.program_id(1)
    scores = jax.lax.dot_general(
        u_ref[...], c_ref[...], (((1,), (1,)), ((), ())),
        preferred_element_type=jnp.float32)
    scores = scores + eb_ref[...]
    m = jnp.max(scores, axis=-1, keepdims=True)
    ex = jnp.exp(scores - m)
    s = ex / jnp.sum(ex, axis=-1, keepdims=True)
    iota = jax.lax.broadcasted_iota(jnp.int32, s.shape, 1)
    e0 = jnp.argmax(s, axis=-1)
    oh0 = iota == e0[:, None]
    e1 = jnp.argmax(jnp.where(oh0, -jnp.inf, s), axis=-1)
    oh1 = iota == e1[:, None]
    ohf = jnp.where(sl == 0, oh0.astype(jnp.float32), oh1.astype(jnp.float32))
    wh = s * ohf
    w = jnp.sum(wh, axis=-1, keepdims=True)
    us_ref[...] = w * u_ref[...]
    oh_ref[...] = ohf
    wh_ref[...] = wh


def _routing_body(oh_ref, dest_ref, bexp_ref, *, blk_m, nblocks):
    P, NR = oh_ref.shape
    CH = 512
    nch = P // CH
    r_i = jax.lax.broadcasted_iota(jnp.int32, (CH, CH), 0)
    c_i = jax.lax.broadcasted_iota(jnp.int32, (CH, CH), 1)
    tri = (r_i >= c_i).astype(jnp.float32)
    carry = jnp.zeros((1, NR), jnp.float32)
    cums = []
    for k in range(nch):
        blk = oh_ref[pl.ds(k * CH, CH), :]
        cb = jax.lax.dot_general(
            tri, blk, (((1,), (0,)), ((), ())),
            precision=HI, preferred_element_type=jnp.float32) + carry
        cums.append(cb)
        carry = cb[CH - 1:CH, :]
    counts = carry
    pc = jnp.ceil(counts * (1.0 / blk_m)) * blk_m
    a_i = jax.lax.broadcasted_iota(jnp.int32, (NR, NR), 0)
    b_i = jax.lax.broadcasted_iota(jnp.int32, (NR, NR), 1)
    lt = (a_i < b_i).astype(jnp.float32)
    off = jax.lax.dot_general(
        pc, lt, (((1,), (0,)), ((), ())),
        precision=HI, preferred_element_type=jnp.float32)
    for k in range(nch):
        ohk = oh_ref[pl.ds(k * CH, CH), :]
        destf = jnp.sum(ohk * (off + cums[k] - 1.0), axis=-1, keepdims=True)
        dest_ref[pl.ds(k * CH, CH), :] = destf.astype(jnp.int32)
    offi = off + pc
    lane = jax.lax.broadcasted_iota(
        jnp.int32, bexp_ref.shape, 1).astype(jnp.float32) * blk_m
    acc = jnp.zeros(bexp_ref.shape, jnp.float32)
    for e in range(NR):
        acc += jnp.where(lane >= offi[0, e], 1.0, 0.0)
    bexp_ref[...] = jnp.minimum(acc, NR - 1).astype(jnp.int32)


def _gmm_body(bexp_sref, x_ref, w_ref, o_ref):
    del bexp_sref
    o_ref[...] = jax.lax.dot_general(
        x_ref[...].astype(jnp.bfloat16), w_ref[0].astype(jnp.bfloat16),
        (((1,), (1,)), ((), ())),
        preferred_element_type=jnp.float32)


def _base_body(u_ref, wsm_ref, bsm_ref, wh0_ref, wh1_ref, br_ref, o_ref):
    ub = u_ref[...]
    y = jax.lax.dot_general(
        ub.astype(jnp.bfloat16), wsm_ref[...].astype(jnp.bfloat16),
        (((1,), (1,)), ((), ())),
        preferred_element_type=jnp.float32)
    whs = wh0_ref[...] + wh1_ref[...]
    bias = jax.lax.dot_general(
        whs, br_ref[...], (((1,), (0,)), ((), ())),
        precision=HI, preferred_element_type=jnp.float32)
    o_ref[...] = ub + y + bsm_ref[...] + bias


def _sc_dispatch(us, dest3d, R):
    P, D = us.shape
    NW, C, G = dest3d.shape
    ppw = P // NW
    mesh = plsc.VectorSubcoreMesh(core_axis_name="c", subcore_axis_name="s")

    @functools.partial(
        pl.kernel, mesh=mesh,
        out_type=jax.ShapeDtypeStruct((R, D), jnp.float32),
        scratch_types=[
            pltpu.VMEM((C, G), jnp.int32),
            pltpu.VMEM((G, D), jnp.float32),
            pltpu.VMEM((G, D), jnp.float32),
            pltpu.SemaphoreType.DMA,
            pltpu.SemaphoreType.DMA,
            pltpu.SemaphoreType.DMA,
            pltpu.SemaphoreType.DMA,
        ])
    def k(us_hbm, dest_hbm, out_hbm, idx_v, rows_a, rows_b,
          sin_a, sin_b, sout_a, sout_b):
        wid = lax.axis_index("s") * 2 + lax.axis_index("c")
        pltpu.sync_copy(dest_hbm.at[wid], idx_v)
        base = wid * ppw
        rows = [rows_a, rows_b]
        sin = [sin_a, sin_b]
        sout = [sout_a, sout_b]
        h_in = [None, None]
        h_out = [None, None]

        def start_in(j):
            ks = j % 2
            h_in[ks] = pltpu.async_copy(
                us_hbm.at[pl.ds(base + j * G, G)], rows[ks], sin[ks])

        start_in(0)
        for j in range(C):
            ks = j % 2
            h_in[ks].wait()
            h_out[ks] = pltpu.async_copy(
                rows[ks], out_hbm.at[idx_v.at[j]], sout[ks])
            if j + 1 < C:
                other = (j + 1) % 2
                if h_out[other] is not None:
                    h_out[other].wait()
                    h_out[other] = None
                start_in(j + 1)
        for ks in range(2):
            if h_out[ks] is not None:
                h_out[ks].wait()

    return k(us, dest3d)


def _sc_combine(base, rows, pos0_3d, pos1_3d):
    T, D = base.shape
    NW, C2, G2 = pos0_3d.shape
    tpw = T // NW
    mesh = plsc.VectorSubcoreMesh(core_axis_name="c", subcore_axis_name="s")

    @functools.partial(
        pl.kernel, mesh=mesh,
        out_type=jax.ShapeDtypeStruct((T, D), jnp.float32),
        scratch_types=[
            pltpu.VMEM((C2, G2), jnp.int32),
            pltpu.VMEM((C2, G2), jnp.int32),
            pltpu.VMEM((G2, D), jnp.float32),
            pltpu.VMEM((G2, D), jnp.float32),
            pltpu.VMEM((G2, D), jnp.float32),
            pltpu.VMEM((G2, D), jnp.float32),
            pltpu.VMEM((G2, D), jnp.float32),
            pltpu.VMEM((G2, D), jnp.float32),
            pltpu.SemaphoreType.DMA,
            pltpu.SemaphoreType.DMA,
        ])
    def k(base_hbm, rows_hbm, p0_hbm, p1_hbm, out_hbm,
          i0_v, i1_v, b_a, b_b, r0_a, r0_b, r1_a, r1_b, sem_a, sem_b):
        wid = lax.axis_index("s") * 2 + lax.axis_index("c")
        pltpu.sync_copy(p0_hbm.at[wid], i0_v)
        pltpu.sync_copy(p1_hbm.at[wid], i1_v)
        tb = wid * tpw
        bufs = [(b_a, r0_a, r1_a, sem_a), (b_b, r0_b, r1_b, sem_b)]
        handles = [None, None]

        def start(j):
            ks = j % 2
            b_v, r0_v, r1_v, sem = bufs[ks]
            handles[ks] = (
                pltpu.async_copy(
                    base_hbm.at[pl.ds(tb + j * G2, G2)], b_v, sem),
                pltpu.async_copy(rows_hbm.at[i0_v.at[j]], r0_v, sem),
                pltpu.async_copy(rows_hbm.at[i1_v.at[j]], r1_v, sem),
            )

        start(0)
        for j in range(C2):
            ks = j % 2
            b_v, r0_v, r1_v, sem = bufs[ks]
            for h in handles[ks]:
                h.wait()
            if j + 1 < C2:
                start(j + 1)

            def row(i, car):
                def col(kk, car2):
                    sl = pl.ds(kk * 16, 16)
                    b_v[i, sl] = b_v[i, sl] + r0_v[i, sl] + r1_v[i, sl]
                    return car2
                return lax.fori_loop(0, D // 16, col, car, unroll=8)

            lax.fori_loop(0, G2, row, 0)
            pltpu.sync_copy(b_v, out_hbm.at[pl.ds(tb + j * G2, G2)])

    return k(base, rows, pos0_3d, pos1_3d)


def kernel(u, centroids, expert_biases, W_r, b_r, W_s, b_s):
    B, S, D = u.shape
    NR = W_r.shape[0]
    T = B * S
    P = 2 * T
    M = 256
    NB = P // M + NR
    R = NB * M
    NW = 32
    G, G2 = 16, 8
    C = P // NW // G
    C2 = T // NW // G2

    u2 = u.reshape(T, D)
    Wsm = jnp.mean(W_s, axis=0)
    bsm = jnp.mean(b_s, axis=0).reshape(1, D)

    MTg = 512
    ngt = T // MTg
    us, ohp, whp = pl.pallas_call(
        _gating_body,
        grid=(ngt, 2),
        in_specs=[
            pl.BlockSpec((MTg, D), lambda i, sl: (i, 0)),
            pl.BlockSpec((NR, D), lambda i, sl: (0, 0)),
            pl.BlockSpec((1, NR), lambda i, sl: (0, 0)),
        ],
        out_specs=[
            pl.BlockSpec((MTg, D), lambda i, sl: (sl * ngt + i, 0)),
            pl.BlockSpec((MTg, NR), lambda i, sl: (sl * ngt + i, 0)),
            pl.BlockSpec((MTg, NR), lambda i, sl: (sl * ngt + i, 0)),
        ],
        out_shape=[
            jax.ShapeDtypeStruct((P, D), jnp.float32),
            jax.ShapeDtypeStruct((P, NR), jnp.float32),
            jax.ShapeDtypeStruct((P, NR), jnp.float32),
        ],
    )(u2, centroids, expert_biases.reshape(1, NR))

    dest, bexp = pl.pallas_call(
        functools.partial(_routing_body, blk_m=M, nblocks=NB),
        grid=(1,),
        in_specs=[pl.BlockSpec((P, NR), lambda i: (0, 0))],
        out_specs=[
            pl.BlockSpec((P, 1), lambda i: (0, 0)),
            pl.BlockSpec((1, 128), lambda i: (0, 0)),
        ],
        out_shape=[
            jax.ShapeDtypeStruct((P, 1), jnp.int32),
            jax.ShapeDtypeStruct((1, 128), jnp.int32),
        ],
    )(ohp)

    dest1 = dest.reshape(P)
    dest3d = dest1.reshape(NW, C, G)
    pos0_3d = dest1[:T].reshape(NW, C2, G2)
    pos1_3d = dest1[T:].reshape(NW, C2, G2)
    bexp_flat = bexp.reshape(128)[:NB]

    xd = _sc_dispatch(us, dest3d, R)

    MTb = 512
    base = pl.pallas_call(
        _base_body,
        grid=(T // MTb,),
        in_specs=[
            pl.BlockSpec((MTb, D), lambda i: (i, 0)),
            pl.BlockSpec((D, D), lambda i: (0, 0)),
            pl.BlockSpec((1, D), lambda i: (0, 0)),
            pl.BlockSpec((MTb, NR), lambda i: (i, 0)),
            pl.BlockSpec((MTb, NR), lambda i: (i + T // MTb, 0)),
            pl.BlockSpec((NR, D), lambda i: (0, 0)),
        ],
        out_specs=pl.BlockSpec((MTb, D), lambda i: (i, 0)),
        out_shape=jax.ShapeDtypeStruct((T, D), jnp.float32),
    )(u2, Wsm, bsm, whp, whp, b_r)

    outs = pl.pallas_call(
        _gmm_body,
        grid_spec=pltpu.PrefetchScalarGridSpec(
            num_scalar_prefetch=1,
            grid=(NB,),
            in_specs=[
                pl.BlockSpec((M, D), lambda i, b: (i, 0)),
                pl.BlockSpec((1, D, D), lambda i, b: (b[i], 0, 0)),
            ],
            out_specs=pl.BlockSpec((M, D), lambda i, b: (i, 0)),
        ),
        out_shape=jax.ShapeDtypeStruct((R, D), jnp.float32),
    )(bexp_flat, xd, W_r)

    h2 = _sc_combine(base, outs, pos0_3d, pos1_3d)
    return h2.reshape(B, S, D)
